# baseline (speedup 1.0000x reference)
import functools

import jax
import jax.numpy as jnp
from jax import lax
from jax.experimental import pallas as pl
from jax.experimental.pallas import tpu as pltpu
from jax.experimental.pallas import tpu_sc as plsc

N_FACTORS = 32
N_ROWS = 1000000
SUP = 128
RPS = SUP // N_FACTORS
BATCH = 16384
NC = 2
NS = 16
NW = NC * NS
BPW = BATCH // NW
CHUNK = 128
NCH = BPW // CHUNK
LANES = 16


def _mf_body(user_r, item_r, uf_r, if_r, out_r,
             uidx, iidx, isup, urows, irows, outv, semu, semi):
    wid = lax.axis_index("s") * NC + lax.axis_index("c")

    pltpu.sync_copy(user_r.at[wid], uidx)
    pltpu.sync_copy(item_r.at[wid], iidx)

    def xf(t, c):
        j = t // (CHUNK // LANES)
        o = (t % (CHUNK // LANES)) * LANES
        isup[j, pl.ds(o, LANES)] = lax.shift_right_logical(
            iidx[j, pl.ds(o, LANES)], 2)
        return c

    lax.fori_loop(0, BPW // LANES, xf, 0)

    lane = lax.iota(jnp.int32, LANES)

    for j in range(NCH):
        ci = pltpu.async_copy(if_r.at[isup.at[j]], irows, semi)

        def issue(p0, c):
            uv = uidx[j, pl.ds(p0 * LANES, LANES)]
            for q in range(LANES):
                p = p0 * LANES + q
                pltpu.async_copy(uf_r.at[pl.ds(uv[q], 1)],
                                 urows.at[pl.ds(p, 1)], semu)
            return c

        lax.fori_loop(0, CHUNK // LANES, issue, 0)

        pltpu.make_async_copy(uf_r.at[pl.ds(0, CHUNK)], urows, semu).wait()
        ci.wait()

        def group(gg, c):
            o = gg * LANES
            co_i = (iidx[j, pl.ds(o, LANES)] & (RPS - 1)) * N_FACTORS
            acc = jnp.zeros((LANES,), jnp.float32)
            for r in range(LANES):
                coi = co_i[r]
                row = o + r
                s0 = urows[row, pl.ds(0, LANES)] * irows[row, pl.ds(coi, LANES)]
                s1 = urows[row, pl.ds(LANES, LANES)] * irows[row, pl.ds(coi + LANES, LANES)]
                tot = jnp.sum(s0 + s1)
                acc = jnp.where(lane == r, tot, acc)
            outv[pl.ds(j * CHUNK + o, LANES)] = acc
            return c

        lax.fori_loop(0, CHUNK // LANES, group, 0)

    pltpu.sync_copy(outv, out_r.at[wid])


_mf = functools.partial(
    pl.kernel,
    mesh=plsc.VectorSubcoreMesh(core_axis_name="c", subcore_axis_name="s"),
    out_type=jax.ShapeDtypeStruct((NW, BPW), jnp.float32),
    scratch_types=[
        pltpu.VMEM((NCH, CHUNK), jnp.int32),
        pltpu.VMEM((NCH, CHUNK), jnp.int32),
        pltpu.VMEM((NCH, CHUNK), jnp.int32),
        pltpu.VMEM((CHUNK, N_FACTORS), jnp.float32),
        pltpu.VMEM((CHUNK, SUP), jnp.float32),
        pltpu.VMEM((BPW,), jnp.float32),
        pltpu.SemaphoreType.DMA,
        pltpu.SemaphoreType.DMA,
    ],
    compiler_params=pltpu.CompilerParams(needs_layout_passes=False),
)(_mf_body)


def kernel(user, item, user_factors, item_factors):
    u = user.astype(jnp.int32).reshape(NW, NCH, CHUNK)
    i = item.astype(jnp.int32).reshape(NW, NCH, CHUNK)
    itf = item_factors.reshape(N_ROWS // RPS, SUP)
    out = _mf(u, i, user_factors, itf)
    return out.reshape(BATCH)
